# baseline (speedup 1.0000x reference)
import functools

import jax
import jax.numpy as jnp
from jax.experimental import pallas as pl
from jax.experimental.pallas import tpu as pltpu

_EPS = 1e-4


def _fused(cu_ref, xc_ref, xp_ref, wq_ref, wk_ref, wm_ref,
           out_ref, qc_ref, zc_ref, pring_ref, yring_ref, *, blk, d, nseg):
    i = pl.program_id(0)

    sl_r = jax.lax.rem(i + 1, 2)
    pj = pring_ref[pl.ds(sl_r, 1), :, :].reshape(blk, 1)
    yj = yring_ref[pl.ds(sl_r, 1), :, :].reshape(blk, d)
    qcarry = qc_ref[...]

    x = xc_ref[...]
    q = jnp.dot(x, wq_ref[...], preferred_element_type=jnp.float32)
    k = jnp.dot(x, wk_ref[...], preferred_element_type=jnp.float32)
    y = jnp.dot(x, wm_ref[...], preferred_element_type=jnp.float32)

    qprev = jnp.concatenate(
        [jnp.where(i == 0, 0.0, qcarry), q[:-1, :]], axis=0)
    qc_ref[...] = q[-1:, :]

    num = jnp.sum(qprev * k, axis=1, keepdims=True)
    qn = jnp.sqrt(jnp.sum(qprev * qprev, axis=1, keepdims=True))
    kn = jnp.sqrt(jnp.sum(k * k, axis=1, keepdims=True))
    cos = num / (qn * kn + 1e-6)
    p_raw = jnp.clip((1.0 - cos) / 2.0, 0.0, 1.0)

    sl_w = jax.lax.rem(i, 2)
    pring_ref[pl.ds(sl_w, 1), :, :] = p_raw[None]
    yring_ref[pl.ds(sl_w, 1), :, :] = y[None]

    ids = (i - 1) * blk + jax.lax.broadcasted_iota(jnp.int32, (blk, 1), 0)
    isf = ids == cu_ref[0]
    for s in range(1, nseg):
        isf = jnp.logical_or(isf, ids == cu_ref[s])
    p = jnp.where(isf, 1.0, pj)
    p = jnp.clip(p, _EPS, 1.0 - _EPS)
    sel = p >= 0.5

    a = jnp.where(isf, 0.0, jnp.where(sel, 1.0 - p, 1.0))
    b = jnp.where(sel, p, 0.0) * yj

    ri = jax.lax.broadcasted_iota(jnp.int32, (blk, blk), 0)
    ci = jax.lax.broadcasted_iota(jnp.int32, (blk, blk), 1)
    row0 = jax.lax.broadcasted_iota(jnp.int32, (blk, 1), 0) == 0
    la = jnp.where(row0, 0.0, jnp.maximum(jnp.log(a), -50.0))
    s_col = la
    step = 1
    while step < blk:
        s_col = s_col + jnp.concatenate(
            [jnp.zeros((step, 1), jnp.float32), s_col[:-step, :]], axis=0)
        step *= 2
    s_row = s_col.reshape((1, blk))
    lmat = jnp.where(ci <= ri, jnp.exp(s_col - s_row), 0.0)
    carry_coef = lmat[:, 0:1] * a[0:1, 0:1]
    z = jnp.dot(lmat, b, preferred_element_type=jnp.float32)
    z = z + carry_coef * zc_ref[...]
    zc_ref[...] = jnp.where(i == 0, 0.0, z[-1:, :])
    out_ref[...] = xp_ref[...] + z


def kernel(flat, cu_seqlens, Wq, Wk, W_main):
    n, d = flat.shape
    blk = 256
    nblk = n // blk
    return pl.pallas_call(
        functools.partial(_fused, blk=blk, d=d, nseg=cu_seqlens.shape[0] - 1),
        grid=(nblk + 1,),
        in_specs=[
            pl.BlockSpec(memory_space=pltpu.SMEM),
            pl.BlockSpec((blk, d), lambda i: (jnp.minimum(i, nblk - 1), 0)),
            pl.BlockSpec((blk, d), lambda i: (jnp.maximum(i - 1, 0), 0)),
            pl.BlockSpec((d, d), lambda i: (0, 0)),
            pl.BlockSpec((d, d), lambda i: (0, 0)),
            pl.BlockSpec((d, d), lambda i: (0, 0)),
        ],
        out_specs=pl.BlockSpec((blk, d), lambda i: (jnp.maximum(i - 1, 0), 0)),
        out_shape=jax.ShapeDtypeStruct((n, d), jnp.float32),
        scratch_shapes=[
            pltpu.VMEM((1, d), jnp.float32),
            pltpu.VMEM((1, d), jnp.float32),
            pltpu.VMEM((2, blk, 1), jnp.float32),
            pltpu.VMEM((2, blk, d), jnp.float32),
        ],
    )(cu_seqlens, flat, flat, Wq, Wk, W_main)


# speedup vs baseline: 1.2230x; 1.2230x over previous
import functools

import jax
import jax.numpy as jnp
from jax.experimental import pallas as pl
from jax.experimental.pallas import tpu as pltpu

_EPS = 1e-4


def _fused(cu_ref, xc_ref, xp_ref, wq_ref, wk_ref, wm_ref,
           out_ref, qc_ref, zc_ref, pring_ref, yring_ref, *, blk, sub, d,
           nseg):
    i = pl.program_id(0)

    sl_r = jax.lax.rem(i + 1, 2)
    pj = pring_ref[pl.ds(sl_r, 1), :, :].reshape(blk, 1)
    yj = yring_ref[pl.ds(sl_r, 1), :, :].reshape(blk, d)
    qcarry = qc_ref[...]

    x = xc_ref[...]
    q = jnp.dot(x, wq_ref[...], preferred_element_type=jnp.float32)
    k = jnp.dot(x, wk_ref[...], preferred_element_type=jnp.float32)
    y = jnp.dot(x, wm_ref[...], preferred_element_type=jnp.float32)

    qprev = jnp.concatenate(
        [jnp.where(i == 0, 0.0, qcarry), q[:-1, :]], axis=0)
    qc_ref[...] = q[-1:, :]

    num = jnp.sum(qprev * k, axis=1, keepdims=True)
    qn = jnp.sqrt(jnp.sum(qprev * qprev, axis=1, keepdims=True))
    kn = jnp.sqrt(jnp.sum(k * k, axis=1, keepdims=True))
    cos = num / (qn * kn + 1e-6)
    p_raw = jnp.clip((1.0 - cos) / 2.0, 0.0, 1.0)

    sl_w = jax.lax.rem(i, 2)
    pring_ref[pl.ds(sl_w, 1), :, :] = p_raw[None]
    yring_ref[pl.ds(sl_w, 1), :, :] = y[None]

    ids = (i - 1) * blk + jax.lax.broadcasted_iota(jnp.int32, (blk, 1), 0)
    isf = ids == cu_ref[0]
    for s in range(1, nseg):
        isf = jnp.logical_or(isf, ids == cu_ref[s])
    p = jnp.where(isf, 1.0, pj)
    p = jnp.clip(p, _EPS, 1.0 - _EPS)
    sel = p >= 0.5

    a = jnp.where(isf, 0.0, jnp.where(sel, 1.0 - p, 1.0))
    b = jnp.where(sel, p, 0.0) * yj

    row0 = jax.lax.broadcasted_iota(jnp.int32, (blk, 1), 0) == 0
    la = jnp.where(row0, 0.0, jnp.maximum(jnp.log(a), -50.0))
    s_col = la
    step = 1
    while step < blk:
        s_col = s_col + jnp.concatenate(
            [jnp.zeros((step, 1), jnp.float32), s_col[:-step, :]], axis=0)
        step *= 2

    ri = jax.lax.broadcasted_iota(jnp.int32, (sub, sub), 0)
    ci = jax.lax.broadcasted_iota(jnp.int32, (sub, sub), 1)
    mask = ci <= ri

    s1 = s_col[:sub, :]
    s2 = s_col[sub:, :]
    lmat1 = jnp.where(mask, jnp.exp(s1 - s1.reshape((1, sub))), 0.0)
    lmat2 = jnp.where(mask, jnp.exp(s2 - s2.reshape((1, sub))), 0.0)
    z1p = jnp.dot(lmat1, b[:sub, :], preferred_element_type=jnp.float32)
    z2p = jnp.dot(lmat2, b[sub:, :], preferred_element_type=jnp.float32)

    zc = zc_ref[...]
    cc1 = lmat1[:, 0:1] * a[0:1, 0:1]
    f2 = jnp.exp(s2 - s1[sub - 1:sub, :])
    z1 = z1p + cc1 * zc
    z2 = z2p + f2 * (z1p[sub - 1:sub, :] + cc1[sub - 1:sub, :] * zc)
    z = jnp.concatenate([z1, z2], axis=0)
    zc_ref[...] = jnp.where(i == 0, 0.0, z[-1:, :])
    out_ref[...] = xp_ref[...] + z


def kernel(flat, cu_seqlens, Wq, Wk, W_main):
    n, d = flat.shape
    blk = 1024
    sub = 512
    nblk = n // blk
    return pl.pallas_call(
        functools.partial(_fused, blk=blk, sub=sub, d=d,
                          nseg=cu_seqlens.shape[0] - 1),
        grid=(nblk + 1,),
        in_specs=[
            pl.BlockSpec(memory_space=pltpu.SMEM),
            pl.BlockSpec((blk, d), lambda i: (jnp.minimum(i, nblk - 1), 0)),
            pl.BlockSpec((blk, d), lambda i: (jnp.maximum(i - 1, 0), 0)),
            pl.BlockSpec((d, d), lambda i: (0, 0)),
            pl.BlockSpec((d, d), lambda i: (0, 0)),
            pl.BlockSpec((d, d), lambda i: (0, 0)),
        ],
        out_specs=pl.BlockSpec((blk, d), lambda i: (jnp.maximum(i - 1, 0), 0)),
        out_shape=jax.ShapeDtypeStruct((n, d), jnp.float32),
        scratch_shapes=[
            pltpu.VMEM((1, d), jnp.float32),
            pltpu.VMEM((1, d), jnp.float32),
            pltpu.VMEM((2, blk, 1), jnp.float32),
            pltpu.VMEM((2, blk, d), jnp.float32),
        ],
    )(cu_seqlens, flat, flat, Wq, Wk, W_main)
